# Initial kernel scaffold; baseline (speedup 1.0000x reference)
import jax
import jax.numpy as jnp
from jax.experimental import pallas as pl


def kernel(nodes, edges, senders, receivers, W1e, b1e, g1e, be1e, W2e, b2e, W1n, b1n, g1n, be1n, W2n, b2n):
    raise NotImplementedError("write your pallas kernel here")


# speedup vs baseline: 2.2356x; 2.2356x over previous
import functools

import jax
import jax.numpy as jnp
from jax import lax
from jax.experimental import pallas as pl
from jax.experimental.pallas import tpu as pltpu
from jax.experimental.pallas import tpu_sc as plsc

_NC = 2
_NS = 16
_NW = _NC * _NS


def _sc_gather(nodes, senders):
    n, d = nodes.shape
    e = senders.shape[0]
    ew = e // _NW
    gk = 40
    mesh = plsc.VectorSubcoreMesh(core_axis_name="c", subcore_axis_name="s")

    @functools.partial(
        pl.kernel,
        out_type=jax.ShapeDtypeStruct((e, d), jnp.float32),
        mesh=mesh,
        scratch_types=[
            pltpu.VMEM((gk,), jnp.int32),
            pltpu.VMEM((gk, d), jnp.float32),
            pltpu.SemaphoreType.DMA,
        ],
    )
    def k(nodes_hbm, senders_hbm, out_hbm, idx_v, rows_v, sem):
        wid = lax.axis_index("s") * _NC + lax.axis_index("c")

        def body(j, carry):
            base = pl.multiple_of(wid * ew + j * gk, 8)
            pltpu.sync_copy(senders_hbm.at[pl.ds(base, gk)], idx_v)
            pltpu.async_copy(nodes_hbm.at[idx_v], rows_v, sem).wait()
            pltpu.sync_copy(rows_v, out_hbm.at[pl.ds(base, gk)])
            return carry

        lax.fori_loop(0, ew // gk, body, 0)

    return k(nodes, senders)


def _sc_scatter(messages, receivers, n):
    e, h = messages.shape
    ew = e // _NW
    sk = 200
    cw = 128
    np_ = h // cw
    rps = 632
    rlast = n - (_NS - 1) * rps

    skc = 40

    z128 = jnp.zeros((rps, cw), jnp.float32)
    e1 = jnp.zeros((skc, cw), jnp.float32).at[:, 0].set(1.0)

    mesh = plsc.VectorSubcoreMesh(core_axis_name="c", subcore_axis_name="s")

    @functools.partial(
        pl.kernel,
        out_type=(
            jax.ShapeDtypeStruct((_NC, n, h), jnp.float32),
            jax.ShapeDtypeStruct((_NC, n, cw), jnp.float32),
        ),
        mesh=mesh,
        scratch_types=[
            pltpu.VMEM((sk, cw), jnp.float32),
            pltpu.VMEM((skc, cw), jnp.float32),
            pltpu.VMEM((sk,), jnp.int32),
            pltpu.VMEM((skc,), jnp.int32),
            pltpu.VMEM_SHARED((n, cw), jnp.float32),
        ],
    )
    def k(msgs_hbm, recv_hbm, z128_hbm, e1_hbm, agg_out, cnt_out,
          msg_v, e1_v, idx_v, idxc_v, agg_s):
        c = lax.axis_index("c")
        s = lax.axis_index("s")
        wid = s * _NC + c
        row0 = pl.multiple_of(s * rps, 8)
        is_last = s == _NS - 1
        pltpu.sync_copy(e1_hbm, e1_v)

        for p in range(np_ + 1):
            @pl.when(jnp.logical_not(is_last))
            def _():
                pltpu.sync_copy(z128_hbm, agg_s.at[pl.ds(row0, rps)])

            @pl.when(is_last)
            def _():
                pltpu.sync_copy(z128_hbm.at[pl.ds(0, rlast)],
                                agg_s.at[pl.ds(row0, rlast)])

            plsc.subcore_barrier()

            if p < np_:
                def body(j, carry):
                    e0 = pl.multiple_of(wid * ew + j * sk, 8)
                    pltpu.sync_copy(recv_hbm.at[pl.ds(e0, sk)], idx_v)
                    pltpu.sync_copy(
                        msgs_hbm.at[pl.ds(e0, sk), pl.ds(p * cw, cw)], msg_v)
                    pltpu.sync_copy(msg_v, agg_s.at[idx_v], add=True)
                    return carry
            else:
                def body(j, carry):
                    e0 = pl.multiple_of(wid * ew + j * skc, 8)
                    pltpu.sync_copy(recv_hbm.at[pl.ds(e0, skc)], idxc_v)
                    pltpu.sync_copy(e1_v, agg_s.at[idxc_v], add=True)
                    return carry

            lax.fori_loop(0, ew // (sk if p < np_ else skc), body, 0)
            plsc.subcore_barrier()

            @pl.when(jnp.logical_not(is_last))
            def _():
                if p < np_:
                    pltpu.sync_copy(
                        agg_s.at[pl.ds(row0, rps)],
                        agg_out.at[c, pl.ds(row0, rps), pl.ds(p * cw, cw)])
                else:
                    pltpu.sync_copy(agg_s.at[pl.ds(row0, rps)],
                                    cnt_out.at[c, pl.ds(row0, rps)])

            @pl.when(is_last)
            def _():
                if p < np_:
                    pltpu.sync_copy(
                        agg_s.at[pl.ds(row0, rlast)],
                        agg_out.at[c, pl.ds(row0, rlast), pl.ds(p * cw, cw)])
                else:
                    pltpu.sync_copy(agg_s.at[pl.ds(row0, rlast)],
                                    cnt_out.at[c, pl.ds(row0, rlast)])

            plsc.subcore_barrier()

    return k(messages, receivers, z128, e1)


def _layer_norm_in_kernel(h, g, b):
    mu = jnp.mean(h, axis=-1, keepdims=True)
    var = jnp.mean((h - mu) * (h - mu), axis=-1, keepdims=True)
    return (h - mu) * lax.rsqrt(var + 1e-6) * g + b


def _tc_edge_mlp(sf, edges, w1t, w1b, b1, g1, be1, w2, b2):
    e, d = sf.shape
    de = edges.shape[1]
    hdim = w2.shape[1]
    be_blk = 1280
    grid = (e // be_blk,)

    def body(sf_ref, ed_ref, w1t_ref, w1b_ref, b1_ref, g1_ref, be1_ref,
             w2_ref, b2_ref, out_ref):
        h = jnp.dot(sf_ref[...], w1t_ref[...],
                    preferred_element_type=jnp.float32)
        h = h + jnp.dot(ed_ref[...], w1b_ref[...],
                        preferred_element_type=jnp.float32)
        h = h + b1_ref[...]
        h = _layer_norm_in_kernel(h, g1_ref[...], be1_ref[...])
        h = jnp.maximum(h, 0.0)
        out_ref[...] = jnp.dot(h, w2_ref[...],
                               preferred_element_type=jnp.float32) + b2_ref[...]

    hsz = w1t.shape[1]
    return pl.pallas_call(
        body,
        grid=grid,
        in_specs=[
            pl.BlockSpec((be_blk, d), lambda i: (i, 0)),
            pl.BlockSpec((be_blk, de), lambda i: (i, 0)),
            pl.BlockSpec((d, hsz), lambda i: (0, 0)),
            pl.BlockSpec((de, hsz), lambda i: (0, 0)),
            pl.BlockSpec((1, hsz), lambda i: (0, 0)),
            pl.BlockSpec((1, hsz), lambda i: (0, 0)),
            pl.BlockSpec((1, hsz), lambda i: (0, 0)),
            pl.BlockSpec((hsz, hdim), lambda i: (0, 0)),
            pl.BlockSpec((1, hdim), lambda i: (0, 0)),
        ],
        out_specs=pl.BlockSpec((be_blk, hdim), lambda i: (i, 0)),
        out_shape=jax.ShapeDtypeStruct((e, hdim), jnp.float32),
    )(sf, edges, w1t, w1b, b1, g1, be1, w2, b2)


def _tc_node_mlp(nodes, a0, a1, c0, c1, w1t, w1b, b1, g1, be1, w2, b2):
    n, d = nodes.shape
    hdim = a0.shape[1]
    do = w2.shape[1]
    bn = 1000
    grid = (n // bn,)

    def body(nd_ref, a0_ref, a1_ref, c0_ref, c1_ref, w1t_ref, w1b_ref,
             b1_ref, g1_ref, be1_ref, w2_ref, b2_ref, out_ref):
        cnt = c0_ref[...][:, 0:1] + c1_ref[...][:, 0:1]
        cnt = jnp.maximum(cnt, 1.0)
        agg = (a0_ref[...] + a1_ref[...]) / cnt
        h = jnp.dot(nd_ref[...], w1t_ref[...],
                    preferred_element_type=jnp.float32)
        h = h + jnp.dot(agg, w1b_ref[...],
                        preferred_element_type=jnp.float32)
        h = h + b1_ref[...]
        h = _layer_norm_in_kernel(h, g1_ref[...], be1_ref[...])
        h = jnp.maximum(h, 0.0)
        out = jnp.dot(h, w2_ref[...],
                      preferred_element_type=jnp.float32) + b2_ref[...]
        out_ref[...] = out + nd_ref[...]

    hsz = w1t.shape[1]
    return pl.pallas_call(
        body,
        grid=grid,
        in_specs=[
            pl.BlockSpec((bn, d), lambda i: (i, 0)),
            pl.BlockSpec((bn, hdim), lambda i: (i, 0)),
            pl.BlockSpec((bn, hdim), lambda i: (i, 0)),
            pl.BlockSpec((bn, 128), lambda i: (i, 0)),
            pl.BlockSpec((bn, 128), lambda i: (i, 0)),
            pl.BlockSpec((d, hsz), lambda i: (0, 0)),
            pl.BlockSpec((hdim, hsz), lambda i: (0, 0)),
            pl.BlockSpec((1, hsz), lambda i: (0, 0)),
            pl.BlockSpec((1, hsz), lambda i: (0, 0)),
            pl.BlockSpec((1, hsz), lambda i: (0, 0)),
            pl.BlockSpec((hsz, do), lambda i: (0, 0)),
            pl.BlockSpec((1, do), lambda i: (0, 0)),
        ],
        out_specs=pl.BlockSpec((bn, do), lambda i: (i, 0)),
        out_shape=jax.ShapeDtypeStruct((n, do), jnp.float32),
    )(nodes, a0, a1, c0, c1, w1t, w1b, b1, g1, be1, w2, b2)


def kernel(nodes, edges, senders, receivers, W1e, b1e, g1e, be1e, W2e, b2e,
           W1n, b1n, g1n, be1n, W2n, b2n):
    n, df = nodes.shape
    senders = senders.astype(jnp.int32)
    receivers = receivers.astype(jnp.int32)

    sf = _sc_gather(nodes, senders)
    messages = _tc_edge_mlp(
        sf, edges,
        W1e[:df], W1e[df:],
        b1e[None, :], g1e[None, :], be1e[None, :],
        W2e, b2e[None, :])
    agg_part, cnt_part = _sc_scatter(messages, receivers, n)
    new_nodes = _tc_node_mlp(
        nodes, agg_part[0], agg_part[1], cnt_part[0], cnt_part[1],
        W1n[:df], W1n[df:],
        b1n[None, :], g1n[None, :], be1n[None, :],
        W2n, b2n[None, :])
    return new_nodes


# speedup vs baseline: 2.4994x; 1.1180x over previous
import functools

import jax
import jax.numpy as jnp
from jax import lax
from jax.experimental import pallas as pl
from jax.experimental.pallas import tpu as pltpu
from jax.experimental.pallas import tpu_sc as plsc

_NC = 2
_NS = 16
_NW = _NC * _NS


def _sc_gather(nodes, senders):
    n, d = nodes.shape
    e = senders.shape[0]
    ew = e // _NW
    gk = 200
    nch = ew // gk
    mesh = plsc.VectorSubcoreMesh(core_axis_name="c", subcore_axis_name="s")

    @functools.partial(
        pl.kernel,
        out_type=jax.ShapeDtypeStruct((e, d), jnp.float32),
        mesh=mesh,
        scratch_types=[
            pltpu.VMEM((gk,), jnp.int32),
            pltpu.VMEM((gk,), jnp.int32),
            pltpu.VMEM((gk, d), jnp.float32),
            pltpu.VMEM((gk, d), jnp.float32),
            pltpu.SemaphoreType.DMA,
            pltpu.SemaphoreType.DMA,
            pltpu.SemaphoreType.DMA,
            pltpu.SemaphoreType.DMA,
            pltpu.SemaphoreType.DMA,
            pltpu.SemaphoreType.DMA,
        ],
    )
    def k(nodes_hbm, senders_hbm, out_hbm, idx0_v, idx1_v, rows0_v, rows1_v,
          si0, si1, sg0, sg1, sw0, sw1):
        wid = lax.axis_index("s") * _NC + lax.axis_index("c")
        idx = (idx0_v, idx1_v)
        rows = (rows0_v, rows1_v)
        si = (si0, si1)
        sg = (sg0, sg1)
        sw = (sw0, sw1)

        def base(j):
            return pl.multiple_of(wid * ew + j * gk, 8)

        def load(j):
            b = j % 2
            return pltpu.async_copy(senders_hbm.at[pl.ds(base(j), gk)],
                                    idx[b], si[b])

        def gath(j):
            b = j % 2
            return pltpu.async_copy(nodes_hbm.at[idx[b]],
                                    rows[b], sg[b])

        def wout(j):
            b = j % 2
            return pltpu.async_copy(rows[b],
                                    out_hbm.at[pl.ds(base(j), gk)], sw[b])

        dl = [None] * nch
        dg = [None] * nch
        dw = [None] * nch
        dl[0] = load(0)
        if nch > 1:
            dl[1] = load(1)
        dl[0].wait()
        dg[0] = gath(0)
        for j in range(nch):
            dg[j].wait()
            if j + 1 < nch:
                dl[j + 1].wait()
                if j >= 1:
                    dw[j - 1].wait()
                dg[j + 1] = gath(j + 1)
            dw[j] = wout(j)
            if j + 2 < nch:
                dl[j + 2] = load(j + 2)
        if nch > 1:
            dw[nch - 2].wait()
        dw[nch - 1].wait()

    return k(nodes, senders)


def _sc_scatter(messages, receivers, n):
    e, h = messages.shape
    ew = e // _NW
    sk = 200
    cw = 128
    np_ = h // cw
    rps = 632
    rlast = n - (_NS - 1) * rps

    skc = 40

    z128 = jnp.zeros((rps, cw), jnp.float32)
    e1 = jnp.zeros((skc, cw), jnp.float32).at[:, 0].set(1.0)

    mesh = plsc.VectorSubcoreMesh(core_axis_name="c", subcore_axis_name="s")

    @functools.partial(
        pl.kernel,
        out_type=(
            jax.ShapeDtypeStruct((_NC, n, h), jnp.float32),
            jax.ShapeDtypeStruct((_NC, n, cw), jnp.float32),
        ),
        mesh=mesh,
        scratch_types=[
            pltpu.VMEM((sk, cw), jnp.float32),
            pltpu.VMEM((skc, cw), jnp.float32),
            pltpu.VMEM((sk,), jnp.int32),
            pltpu.VMEM((skc,), jnp.int32),
            pltpu.VMEM_SHARED((n, cw), jnp.float32),
        ],
    )
    def k(msgs_hbm, recv_hbm, z128_hbm, e1_hbm, agg_out, cnt_out,
          msg_v, e1_v, idx_v, idxc_v, agg_s):
        c = lax.axis_index("c")
        s = lax.axis_index("s")
        wid = s * _NC + c
        row0 = pl.multiple_of(s * rps, 8)
        is_last = s == _NS - 1
        pltpu.sync_copy(e1_hbm, e1_v)

        for p in range(np_ + 1):
            @pl.when(jnp.logical_not(is_last))
            def _():
                pltpu.sync_copy(z128_hbm, agg_s.at[pl.ds(row0, rps)])

            @pl.when(is_last)
            def _():
                pltpu.sync_copy(z128_hbm.at[pl.ds(0, rlast)],
                                agg_s.at[pl.ds(row0, rlast)])

            plsc.subcore_barrier()

            if p < np_:
                def body(j, carry):
                    e0 = pl.multiple_of(wid * ew + j * sk, 8)
                    pltpu.sync_copy(recv_hbm.at[pl.ds(e0, sk)], idx_v)
                    pltpu.sync_copy(
                        msgs_hbm.at[pl.ds(e0, sk), pl.ds(p * cw, cw)], msg_v)
                    pltpu.sync_copy(msg_v, agg_s.at[idx_v], add=True)
                    return carry
            else:
                def body(j, carry):
                    e0 = pl.multiple_of(wid * ew + j * skc, 8)
                    pltpu.sync_copy(recv_hbm.at[pl.ds(e0, skc)], idxc_v)
                    pltpu.sync_copy(e1_v, agg_s.at[idxc_v], add=True)
                    return carry

            lax.fori_loop(0, ew // (sk if p < np_ else skc), body, 0)
            plsc.subcore_barrier()

            @pl.when(jnp.logical_not(is_last))
            def _():
                if p < np_:
                    pltpu.sync_copy(
                        agg_s.at[pl.ds(row0, rps)],
                        agg_out.at[c, pl.ds(row0, rps), pl.ds(p * cw, cw)])
                else:
                    pltpu.sync_copy(agg_s.at[pl.ds(row0, rps)],
                                    cnt_out.at[c, pl.ds(row0, rps)])

            @pl.when(is_last)
            def _():
                if p < np_:
                    pltpu.sync_copy(
                        agg_s.at[pl.ds(row0, rlast)],
                        agg_out.at[c, pl.ds(row0, rlast), pl.ds(p * cw, cw)])
                else:
                    pltpu.sync_copy(agg_s.at[pl.ds(row0, rlast)],
                                    cnt_out.at[c, pl.ds(row0, rlast)])

            plsc.subcore_barrier()

    return k(messages, receivers, z128, e1)


def _layer_norm_in_kernel(h, g, b):
    mu = jnp.mean(h, axis=-1, keepdims=True)
    var = jnp.mean((h - mu) * (h - mu), axis=-1, keepdims=True)
    return (h - mu) * lax.rsqrt(var + 1e-6) * g + b


def _tc_edge_mlp(sf, edges, w1t, w1b, b1, g1, be1, w2, b2):
    e, d = sf.shape
    de = edges.shape[1]
    hdim = w2.shape[1]
    be_blk = 1280
    grid = (e // be_blk,)

    def body(sf_ref, ed_ref, w1t_ref, w1b_ref, b1_ref, g1_ref, be1_ref,
             w2_ref, b2_ref, out_ref):
        h = jnp.dot(sf_ref[...].astype(jnp.bfloat16), w1t_ref[...],
                    preferred_element_type=jnp.float32)
        h = h + jnp.dot(ed_ref[...].astype(jnp.bfloat16), w1b_ref[...],
                        preferred_element_type=jnp.float32)
        h = h + b1_ref[...]
        h = _layer_norm_in_kernel(h, g1_ref[...], be1_ref[...])
        h = jnp.maximum(h, 0.0)
        out_ref[...] = jnp.dot(h.astype(jnp.bfloat16), w2_ref[...],
                               preferred_element_type=jnp.float32) + b2_ref[...]

    hsz = w1t.shape[1]
    return pl.pallas_call(
        body,
        grid=grid,
        in_specs=[
            pl.BlockSpec((be_blk, d), lambda i: (i, 0)),
            pl.BlockSpec((be_blk, de), lambda i: (i, 0)),
            pl.BlockSpec((d, hsz), lambda i: (0, 0)),
            pl.BlockSpec((de, hsz), lambda i: (0, 0)),
            pl.BlockSpec((1, hsz), lambda i: (0, 0)),
            pl.BlockSpec((1, hsz), lambda i: (0, 0)),
            pl.BlockSpec((1, hsz), lambda i: (0, 0)),
            pl.BlockSpec((hsz, hdim), lambda i: (0, 0)),
            pl.BlockSpec((1, hdim), lambda i: (0, 0)),
        ],
        out_specs=pl.BlockSpec((be_blk, hdim), lambda i: (i, 0)),
        out_shape=jax.ShapeDtypeStruct((e, hdim), jnp.float32),
    )(sf, edges, w1t, w1b, b1, g1, be1, w2, b2)


def _tc_node_mlp(nodes, a0, a1, c0, c1, w1t, w1b, b1, g1, be1, w2, b2):
    n, d = nodes.shape
    hdim = a0.shape[1]
    do = w2.shape[1]
    bn = 1000
    grid = (n // bn,)

    def body(nd_ref, a0_ref, a1_ref, c0_ref, c1_ref, w1t_ref, w1b_ref,
             b1_ref, g1_ref, be1_ref, w2_ref, b2_ref, out_ref):
        cnt = c0_ref[...][:, 0:1] + c1_ref[...][:, 0:1]
        cnt = jnp.maximum(cnt, 1.0)
        agg = (a0_ref[...] + a1_ref[...]) / cnt
        h = jnp.dot(nd_ref[...], w1t_ref[...],
                    preferred_element_type=jnp.float32)
        h = h + jnp.dot(agg, w1b_ref[...],
                        preferred_element_type=jnp.float32)
        h = h + b1_ref[...]
        h = _layer_norm_in_kernel(h, g1_ref[...], be1_ref[...])
        h = jnp.maximum(h, 0.0)
        out = jnp.dot(h, w2_ref[...],
                      preferred_element_type=jnp.float32) + b2_ref[...]
        out_ref[...] = out + nd_ref[...]

    hsz = w1t.shape[1]
    return pl.pallas_call(
        body,
        grid=grid,
        in_specs=[
            pl.BlockSpec((bn, d), lambda i: (i, 0)),
            pl.BlockSpec((bn, hdim), lambda i: (i, 0)),
            pl.BlockSpec((bn, hdim), lambda i: (i, 0)),
            pl.BlockSpec((bn, 128), lambda i: (i, 0)),
            pl.BlockSpec((bn, 128), lambda i: (i, 0)),
            pl.BlockSpec((d, hsz), lambda i: (0, 0)),
            pl.BlockSpec((hdim, hsz), lambda i: (0, 0)),
            pl.BlockSpec((1, hsz), lambda i: (0, 0)),
            pl.BlockSpec((1, hsz), lambda i: (0, 0)),
            pl.BlockSpec((1, hsz), lambda i: (0, 0)),
            pl.BlockSpec((hsz, do), lambda i: (0, 0)),
            pl.BlockSpec((1, do), lambda i: (0, 0)),
        ],
        out_specs=pl.BlockSpec((bn, do), lambda i: (i, 0)),
        out_shape=jax.ShapeDtypeStruct((n, do), jnp.float32),
    )(nodes, a0, a1, c0, c1, w1t, w1b, b1, g1, be1, w2, b2)


def kernel(nodes, edges, senders, receivers, W1e, b1e, g1e, be1e, W2e, b2e,
           W1n, b1n, g1n, be1n, W2n, b2n):
    n, df = nodes.shape
    senders = senders.astype(jnp.int32)
    receivers = receivers.astype(jnp.int32)

    sf = _sc_gather(nodes, senders)
    messages = _tc_edge_mlp(
        sf, edges,
        W1e[:df].astype(jnp.bfloat16), W1e[df:].astype(jnp.bfloat16),
        b1e[None, :], g1e[None, :], be1e[None, :],
        W2e.astype(jnp.bfloat16), b2e[None, :])
    agg_part, cnt_part = _sc_scatter(messages, receivers, n)
    new_nodes = _tc_node_mlp(
        nodes, agg_part[0], agg_part[1], cnt_part[0], cnt_part[1],
        W1n[:df], W1n[df:],
        b1n[None, :], g1n[None, :], be1n[None, :],
        W2n, b2n[None, :])
    return new_nodes


# speedup vs baseline: 3.1051x; 1.2423x over previous
import functools

import jax
import jax.numpy as jnp
from jax import lax
from jax.experimental import pallas as pl
from jax.experimental.pallas import tpu as pltpu
from jax.experimental.pallas import tpu_sc as plsc

_NC = 2
_NS = 16
_NW = _NC * _NS


def _sc_gather(nodes, senders):
    n, d = nodes.shape
    e = senders.shape[0]
    ew = e // _NW
    gk = 200
    nch = ew // gk
    mesh = plsc.VectorSubcoreMesh(core_axis_name="c", subcore_axis_name="s")

    @functools.partial(
        pl.kernel,
        out_type=jax.ShapeDtypeStruct((e, d), jnp.float32),
        mesh=mesh,
        scratch_types=[
            pltpu.VMEM((gk,), jnp.int32),
            pltpu.VMEM((gk,), jnp.int32),
            pltpu.VMEM((gk, d), jnp.float32),
            pltpu.VMEM((gk, d), jnp.float32),
            pltpu.SemaphoreType.DMA,
            pltpu.SemaphoreType.DMA,
            pltpu.SemaphoreType.DMA,
            pltpu.SemaphoreType.DMA,
            pltpu.SemaphoreType.DMA,
            pltpu.SemaphoreType.DMA,
        ],
    )
    def k(nodes_hbm, senders_hbm, out_hbm, idx0_v, idx1_v, rows0_v, rows1_v,
          si0, si1, sg0, sg1, sw0, sw1):
        wid = lax.axis_index("s") * _NC + lax.axis_index("c")
        idx = (idx0_v, idx1_v)
        rows = (rows0_v, rows1_v)
        si = (si0, si1)
        sg = (sg0, sg1)
        sw = (sw0, sw1)

        def base(j):
            return pl.multiple_of(wid * ew + j * gk, 8)

        def load(j):
            b = j % 2
            return pltpu.async_copy(senders_hbm.at[pl.ds(base(j), gk)],
                                    idx[b], si[b])

        def gath(j):
            b = j % 2
            return pltpu.async_copy(nodes_hbm.at[idx[b]],
                                    rows[b], sg[b])

        def wout(j):
            b = j % 2
            return pltpu.async_copy(rows[b],
                                    out_hbm.at[pl.ds(base(j), gk)], sw[b])

        dl = [None] * nch
        dg = [None] * nch
        dw = [None] * nch
        dl[0] = load(0)
        if nch > 1:
            dl[1] = load(1)
        dl[0].wait()
        dg[0] = gath(0)
        for j in range(nch):
            dg[j].wait()
            if j + 1 < nch:
                dl[j + 1].wait()
                if j >= 1:
                    dw[j - 1].wait()
                dg[j + 1] = gath(j + 1)
            dw[j] = wout(j)
            if j + 2 < nch:
                dl[j + 2] = load(j + 2)
        if nch > 1:
            dw[nch - 2].wait()
        dw[nch - 1].wait()

    return k(nodes, senders)


def _sc_scatter(messages, receivers, n):
    e, h = messages.shape
    ew = e // _NW
    sk = 192
    nch = ew // sk
    rem = ew - nch * sk
    cw = 128
    np_ = h // cw
    rps = 632
    rlast = n - (_NS - 1) * rps

    z128 = jnp.zeros((rps, cw), jnp.float32)
    e1 = jnp.zeros((sk, cw), jnp.float32).at[:, 0].set(1.0)

    mesh = plsc.VectorSubcoreMesh(core_axis_name="c", subcore_axis_name="s")

    @functools.partial(
        pl.kernel,
        out_type=(
            jax.ShapeDtypeStruct((_NC, n, h), jnp.float32),
            jax.ShapeDtypeStruct((_NC, n, cw), jnp.float32),
        ),
        mesh=mesh,
        scratch_types=[
            pltpu.VMEM((sk, cw), jnp.float32),
            pltpu.VMEM((sk, cw), jnp.float32),
            pltpu.VMEM((rem, cw), jnp.float32),
            pltpu.VMEM((sk,), jnp.int32),
            pltpu.VMEM((sk,), jnp.int32),
            pltpu.VMEM((rem,), jnp.int32),
            pltpu.VMEM_SHARED((n, cw), jnp.float32),
            pltpu.SemaphoreType.DMA,
            pltpu.SemaphoreType.DMA,
            pltpu.SemaphoreType.DMA,
            pltpu.SemaphoreType.DMA,
            pltpu.SemaphoreType.DMA,
        ],
    )
    def k(msgs_hbm, recv_hbm, z128_hbm, e1_hbm, agg_out, cnt_out,
          msg0_v, msg1_v, msge_v, idx0_v, idx1_v, idxe_v, agg_s,
          si0, si1, sm0, sm1, se):
        c = lax.axis_index("c")
        s = lax.axis_index("s")
        wid = s * _NC + c
        row0 = pl.multiple_of(s * rps, 8)
        is_last = s == _NS - 1
        wbase = wid * ew
        msgb = (msg0_v, msg1_v)
        idxb = (idx0_v, idx1_v)
        sib = (si0, si1)
        smb = (sm0, sm1)

        def fire(j, b, p):
            e0 = pl.multiple_of(wbase + j * sk, 8)
            pltpu.async_copy(recv_hbm.at[pl.ds(e0, sk)], idxb[b], sib[b])
            if p < np_:
                pltpu.async_copy(
                    msgs_hbm.at[pl.ds(e0, sk), pl.ds(p * cw, cw)],
                    msgb[b], smb[b])

        def waitld(j, b, p):
            e0 = pl.multiple_of(wbase + j * sk, 8)
            pltpu.make_async_copy(recv_hbm.at[pl.ds(e0, sk)],
                                  idxb[b], sib[b]).wait()
            if p < np_:
                pltpu.make_async_copy(
                    msgs_hbm.at[pl.ds(e0, sk), pl.ds(p * cw, cw)],
                    msgb[b], smb[b]).wait()

        for p in range(np_ + 1):
            @pl.when(jnp.logical_not(is_last))
            def _():
                pltpu.sync_copy(z128_hbm, agg_s.at[pl.ds(row0, rps)])

            @pl.when(is_last)
            def _():
                pltpu.sync_copy(z128_hbm.at[pl.ds(0, rlast)],
                                agg_s.at[pl.ds(row0, rlast)])

            plsc.subcore_barrier()

            if p == np_:
                pltpu.sync_copy(e1_hbm, msg0_v)
                pltpu.sync_copy(e1_hbm.at[pl.ds(0, rem)], msge_v)

            fire(0, 0, p)

            def super(i, carry):
                j0 = 2 * i
                fire(j0 + 1, 1, p)
                waitld(j0, 0, p)
                pltpu.sync_copy(msg0_v, agg_s.at[idx0_v], add=True)

                @pl.when(i < nch // 2 - 1)
                def _():
                    fire(j0 + 2, 0, p)

                waitld(j0 + 1, 1, p)
                pltpu.sync_copy(msg1_v if p < np_ else msg0_v,
                                agg_s.at[idx1_v], add=True)
                return carry

            lax.fori_loop(0, nch // 2, super, 0)

            if rem:
                e0r = pl.multiple_of(wbase + nch * sk, 8)
                pltpu.async_copy(recv_hbm.at[pl.ds(e0r, rem)], idxe_v,
                                 se).wait()
                if p < np_:
                    pltpu.async_copy(
                        msgs_hbm.at[pl.ds(e0r, rem), pl.ds(p * cw, cw)],
                        msge_v, se).wait()
                pltpu.sync_copy(msge_v, agg_s.at[idxe_v], add=True)
            plsc.subcore_barrier()

            @pl.when(jnp.logical_not(is_last))
            def _():
                if p < np_:
                    pltpu.sync_copy(
                        agg_s.at[pl.ds(row0, rps)],
                        agg_out.at[c, pl.ds(row0, rps), pl.ds(p * cw, cw)])
                else:
                    pltpu.sync_copy(agg_s.at[pl.ds(row0, rps)],
                                    cnt_out.at[c, pl.ds(row0, rps)])

            @pl.when(is_last)
            def _():
                if p < np_:
                    pltpu.sync_copy(
                        agg_s.at[pl.ds(row0, rlast)],
                        agg_out.at[c, pl.ds(row0, rlast), pl.ds(p * cw, cw)])
                else:
                    pltpu.sync_copy(agg_s.at[pl.ds(row0, rlast)],
                                    cnt_out.at[c, pl.ds(row0, rlast)])

            plsc.subcore_barrier()

    return k(messages, receivers, z128, e1)


def _layer_norm_in_kernel(h, g, b):
    mu = jnp.mean(h, axis=-1, keepdims=True)
    var = jnp.mean((h - mu) * (h - mu), axis=-1, keepdims=True)
    return (h - mu) * lax.rsqrt(var + 1e-6) * g + b


def _tc_edge_mlp(sf, edges, w1t, w1b, b1, g1, be1, w2, b2):
    e, d = sf.shape
    de = edges.shape[1]
    hdim = w2.shape[1]
    be_blk = 1280
    grid = (e // be_blk,)

    def body(sf_ref, ed_ref, w1t_ref, w1b_ref, b1_ref, g1_ref, be1_ref,
             w2_ref, b2_ref, out_ref):
        h = jnp.dot(sf_ref[...].astype(jnp.bfloat16), w1t_ref[...],
                    preferred_element_type=jnp.float32)
        h = h + jnp.dot(ed_ref[...].astype(jnp.bfloat16), w1b_ref[...],
                        preferred_element_type=jnp.float32)
        h = h + b1_ref[...]
        h = _layer_norm_in_kernel(h, g1_ref[...], be1_ref[...])
        h = jnp.maximum(h, 0.0)
        out_ref[...] = jnp.dot(h.astype(jnp.bfloat16), w2_ref[...],
                               preferred_element_type=jnp.float32) + b2_ref[...]

    hsz = w1t.shape[1]
    return pl.pallas_call(
        body,
        grid=grid,
        in_specs=[
            pl.BlockSpec((be_blk, d), lambda i: (i, 0)),
            pl.BlockSpec((be_blk, de), lambda i: (i, 0)),
            pl.BlockSpec((d, hsz), lambda i: (0, 0)),
            pl.BlockSpec((de, hsz), lambda i: (0, 0)),
            pl.BlockSpec((1, hsz), lambda i: (0, 0)),
            pl.BlockSpec((1, hsz), lambda i: (0, 0)),
            pl.BlockSpec((1, hsz), lambda i: (0, 0)),
            pl.BlockSpec((hsz, hdim), lambda i: (0, 0)),
            pl.BlockSpec((1, hdim), lambda i: (0, 0)),
        ],
        out_specs=pl.BlockSpec((be_blk, hdim), lambda i: (i, 0)),
        out_shape=jax.ShapeDtypeStruct((e, hdim), jnp.float32),
    )(sf, edges, w1t, w1b, b1, g1, be1, w2, b2)


def _tc_node_mlp(nodes, a0, a1, c0, c1, w1t, w1b, b1, g1, be1, w2, b2):
    n, d = nodes.shape
    hdim = a0.shape[1]
    do = w2.shape[1]
    bn = 1000
    grid = (n // bn,)

    def body(nd_ref, a0_ref, a1_ref, c0_ref, c1_ref, w1t_ref, w1b_ref,
             b1_ref, g1_ref, be1_ref, w2_ref, b2_ref, out_ref):
        cnt = c0_ref[...][:, 0:1] + c1_ref[...][:, 0:1]
        cnt = jnp.maximum(cnt, 1.0)
        agg = (a0_ref[...] + a1_ref[...]) / cnt
        h = jnp.dot(nd_ref[...], w1t_ref[...],
                    preferred_element_type=jnp.float32)
        h = h + jnp.dot(agg, w1b_ref[...],
                        preferred_element_type=jnp.float32)
        h = h + b1_ref[...]
        h = _layer_norm_in_kernel(h, g1_ref[...], be1_ref[...])
        h = jnp.maximum(h, 0.0)
        out = jnp.dot(h, w2_ref[...],
                      preferred_element_type=jnp.float32) + b2_ref[...]
        out_ref[...] = out + nd_ref[...]

    hsz = w1t.shape[1]
    return pl.pallas_call(
        body,
        grid=grid,
        in_specs=[
            pl.BlockSpec((bn, d), lambda i: (i, 0)),
            pl.BlockSpec((bn, hdim), lambda i: (i, 0)),
            pl.BlockSpec((bn, hdim), lambda i: (i, 0)),
            pl.BlockSpec((bn, 128), lambda i: (i, 0)),
            pl.BlockSpec((bn, 128), lambda i: (i, 0)),
            pl.BlockSpec((d, hsz), lambda i: (0, 0)),
            pl.BlockSpec((hdim, hsz), lambda i: (0, 0)),
            pl.BlockSpec((1, hsz), lambda i: (0, 0)),
            pl.BlockSpec((1, hsz), lambda i: (0, 0)),
            pl.BlockSpec((1, hsz), lambda i: (0, 0)),
            pl.BlockSpec((hsz, do), lambda i: (0, 0)),
            pl.BlockSpec((1, do), lambda i: (0, 0)),
        ],
        out_specs=pl.BlockSpec((bn, do), lambda i: (i, 0)),
        out_shape=jax.ShapeDtypeStruct((n, do), jnp.float32),
    )(nodes, a0, a1, c0, c1, w1t, w1b, b1, g1, be1, w2, b2)


def kernel(nodes, edges, senders, receivers, W1e, b1e, g1e, be1e, W2e, b2e,
           W1n, b1n, g1n, be1n, W2n, b2n):
    n, df = nodes.shape
    senders = senders.astype(jnp.int32)
    receivers = receivers.astype(jnp.int32)

    sf = _sc_gather(nodes, senders)
    messages = _tc_edge_mlp(
        sf, edges,
        W1e[:df].astype(jnp.bfloat16), W1e[df:].astype(jnp.bfloat16),
        b1e[None, :], g1e[None, :], be1e[None, :],
        W2e.astype(jnp.bfloat16), b2e[None, :])
    agg_part, cnt_part = _sc_scatter(messages, receivers, n)
    new_nodes = _tc_node_mlp(
        nodes, agg_part[0], agg_part[1], cnt_part[0], cnt_part[1],
        W1n[:df], W1n[df:],
        b1n[None, :], g1n[None, :], be1n[None, :],
        W2n, b2n[None, :])
    return new_nodes
